# baseline (speedup 1.0000x reference)
import functools

import jax
import jax.numpy as jnp
from jax.experimental import pallas as pl
from jax.experimental.pallas import tpu as pltpu

COMMITMENT_COST = 0.25


def _vq_body(x_ref, emb_t_ref, emb_aug_t_ref, out_ref, loss_ref, nd_scr,
             *, loss_scale, chunk):
    x = x_ref[0]
    emb_t = emb_t_ref[...]
    e_hat = emb_t / jnp.sqrt(jnp.sum(emb_t * emb_t, axis=0, keepdims=True))
    xsq = jnp.sum(x * x, axis=0, keepdims=True)
    esq = jnp.sum(e_hat * e_hat, axis=0, keepdims=True)
    esq_col = esq.reshape(-1, 1)
    n_embed = emb_t.shape[1]
    m = None
    for c in range(0, n_embed, chunk):
        sl = slice(c, c + chunk)
        s_c = jax.lax.dot_general(
            e_hat[:, sl], x, (((0,), (0,)), ((), ())),
            preferred_element_type=jnp.float32)
        nd_c = (2.0 * s_c - xsq) - esq_col[sl, :]
        nd_scr[sl, :] = nd_c
        m_c = jnp.max(nd_c, axis=0, keepdims=True)
        m = m_c if m is None else jnp.maximum(m, m_c)
    qc = None
    for c in range(0, n_embed, chunk):
        sl = slice(c, c + chunk)
        oh_c = (nd_scr[sl, :] == m).astype(jnp.bfloat16)
        qc_c = jnp.dot(emb_aug_t_ref[:, sl], oh_c,
                       preferred_element_type=jnp.float32)
        qc = qc_c if qc is None else qc + qc_c
    nmel = x.shape[0]
    q = qc[:nmel, :] / qc[nmel:nmel + 1, :]
    d = q - x
    out_ref[0] = x + d
    loss_ref[0, 0, 0] = jnp.sum(d * d) * loss_scale


def kernel(cnt_emb, embedding_weight):
    b, t, d = cnt_emb.shape
    n_embed = embedding_weight.shape[0]
    blk = 4096
    tsteps = t // blk
    loss_scale = (1.0 + COMMITMENT_COST) / (b * t * d)

    x_t = jnp.transpose(cnt_emb, (0, 2, 1))
    emb_t = embedding_weight.T
    emb_aug_t = jnp.concatenate(
        [embedding_weight, jnp.ones((n_embed, 1), jnp.float32)], axis=1
    ).astype(jnp.bfloat16).T

    body = functools.partial(_vq_body, loss_scale=loss_scale, chunk=512)
    out_t, loss = pl.pallas_call(
        body,
        grid=(b, tsteps),
        in_specs=[
            pl.BlockSpec((1, d, blk), lambda i, j: (i, 0, j)),
            pl.BlockSpec((d, n_embed), lambda i, j: (0, 0)),
            pl.BlockSpec((d + 1, n_embed), lambda i, j: (0, 0)),
        ],
        out_specs=[
            pl.BlockSpec((1, d, blk), lambda i, j: (i, 0, j)),
            pl.BlockSpec((1, 1, 1), lambda i, j: (i, j, 0),
                         memory_space=pltpu.SMEM),
        ],
        out_shape=[
            jax.ShapeDtypeStruct((b, d, t), jnp.float32),
            jax.ShapeDtypeStruct((b, tsteps, 1), jnp.float32),
        ],
        scratch_shapes=[pltpu.VMEM((n_embed, blk), jnp.float32)],
    )(x_t, emb_t, emb_aug_t)
    return jnp.transpose(out_t, (0, 2, 1)), jnp.sum(loss)


# speedup vs baseline: 1.0185x; 1.0185x over previous
import functools

import jax
import jax.numpy as jnp
from jax.experimental import pallas as pl
from jax.experimental.pallas import tpu as pltpu

COMMITMENT_COST = 0.25


def _vq_body(x_ref, emb_t_ref, emb_aug_t_ref, out_ref, loss_ref, nd_scr,
             *, loss_scale, chunk):
    x = x_ref[0]
    emb_t = emb_t_ref[...]
    e_hat = emb_t / jnp.sqrt(jnp.sum(emb_t * emb_t, axis=0, keepdims=True))
    xsq = jnp.sum(x * x, axis=0, keepdims=True)
    esq = jnp.sum(e_hat * e_hat, axis=0, keepdims=True)
    esq_col = esq.reshape(-1, 1)
    n_embed = emb_t.shape[1]
    m = None
    for c in range(0, n_embed, chunk):
        sl = slice(c, c + chunk)
        s_c = jax.lax.dot_general(
            e_hat[:, sl], x, (((0,), (0,)), ((), ())),
            preferred_element_type=jnp.float32)
        nd_c = (2.0 * s_c - xsq) - esq_col[sl, :]
        nd_scr[sl, :] = nd_c
        m_c = jnp.max(nd_c, axis=0, keepdims=True)
        m = m_c if m is None else jnp.maximum(m, m_c)
    qc = None
    for c in range(0, n_embed, chunk):
        sl = slice(c, c + chunk)
        oh_c = (nd_scr[sl, :] == m).astype(jnp.bfloat16)
        qc_c = jnp.dot(emb_aug_t_ref[:, sl], oh_c,
                       preferred_element_type=jnp.float32)
        qc = qc_c if qc is None else qc + qc_c
    nmel = x.shape[0]
    q = qc[:nmel, :] / qc[nmel:nmel + 1, :]
    d = q - x
    out_ref[0] = x + d
    loss_ref[0, 0, 0] = jnp.sum(d * d) * loss_scale


def kernel(cnt_emb, embedding_weight):
    b, t, d = cnt_emb.shape
    n_embed = embedding_weight.shape[0]
    blk = 4096
    tsteps = t // blk
    loss_scale = (1.0 + COMMITMENT_COST) / (b * t * d)

    x_t = jnp.transpose(cnt_emb, (0, 2, 1))
    emb_t = embedding_weight.T
    emb_aug_t = jnp.concatenate(
        [embedding_weight, jnp.ones((n_embed, 1), jnp.float32)], axis=1
    ).astype(jnp.bfloat16).T

    body = functools.partial(_vq_body, loss_scale=loss_scale, chunk=256)
    out_t, loss = pl.pallas_call(
        body,
        grid=(b, tsteps),
        in_specs=[
            pl.BlockSpec((1, d, blk), lambda i, j: (i, 0, j)),
            pl.BlockSpec((d, n_embed), lambda i, j: (0, 0)),
            pl.BlockSpec((d + 1, n_embed), lambda i, j: (0, 0)),
        ],
        out_specs=[
            pl.BlockSpec((1, d, blk), lambda i, j: (i, 0, j)),
            pl.BlockSpec((1, 1, 1), lambda i, j: (i, j, 0),
                         memory_space=pltpu.SMEM),
        ],
        out_shape=[
            jax.ShapeDtypeStruct((b, d, t), jnp.float32),
            jax.ShapeDtypeStruct((b, tsteps, 1), jnp.float32),
        ],
        scratch_shapes=[pltpu.VMEM((n_embed, blk), jnp.float32)],
    )(x_t, emb_t, emb_aug_t)
    return jnp.transpose(out_t, (0, 2, 1)), jnp.sum(loss)
